# baseline (speedup 1.0000x reference)
import functools

import jax
import jax.numpy as jnp
from jax import lax
from jax.experimental import pallas as pl
from jax.experimental.pallas import tpu as pltpu
from jax.experimental.pallas import tpu_sc as plsc

_NUM_FIELDS = 26
_VOCAB = 100000
_EMBED_DIM = 32
_NUM_NUMERICAL = 13
_CHUNK = 32
_LINES_PER_VOCAB = _VOCAB // 4


def _build(batch):
    info = plsc.get_sparse_core_info()
    n_workers = info.num_cores * info.num_subcores
    b_per_w = batch // n_workers
    n_chunks = b_per_w // _CHUNK
    emb_d = _NUM_FIELDS * _EMBED_DIM
    mesh = plsc.VectorSubcoreMesh(core_axis_name="c", subcore_axis_name="s")

    @functools.partial(
        pl.kernel,
        mesh=mesh,
        out_type=jax.ShapeDtypeStruct((batch, emb_d), jnp.float32),
        compiler_params=pltpu.CompilerParams(needs_layout_passes=False),
        scratch_types=[
            pltpu.VMEM((_NUM_FIELDS, b_per_w), jnp.int32),
            pltpu.VMEM((_CHUNK,), jnp.int32),
            pltpu.VMEM((_CHUNK,), jnp.int32),
            pltpu.VMEM((_CHUNK,), jnp.int32),
            pltpu.VMEM((_CHUNK, 128), jnp.float32),
            pltpu.VMEM((_CHUNK, 128), jnp.float32),
            pltpu.VMEM((_CHUNK, 128), jnp.float32),
            pltpu.VMEM((_CHUNK, emb_d), jnp.float32),
            pltpu.VMEM((_CHUNK, emb_d), jnp.float32),
            pltpu.SemaphoreType.DMA,
            pltpu.SemaphoreType.DMA,
            pltpu.SemaphoreType.DMA,
            pltpu.SemaphoreType.DMA,
            pltpu.SemaphoreType.DMA,
        ],
    )
    def flattener(idx_hbm, tab_hbm, out_hbm, raw_v, jbuf0, jbuf1, jbuf2,
                  stage0, stage1, stage2, asm0, asm1,
                  gsem0, gsem1, gsem2, wsem0, wsem1):
        jbufs = (jbuf0, jbuf1, jbuf2)
        stages = (stage0, stage1, stage2)
        asms = (asm0, asm1)
        gsems = (gsem0, gsem1, gsem2)
        wsems = (wsem0, wsem1)
        wid = lax.axis_index("s") * info.num_cores + lax.axis_index("c")
        base = wid * b_per_w

        pltpu.sync_copy(
            idx_hbm.at[:, pl.ds(pl.multiple_of(base, b_per_w), b_per_w)],
            raw_v)

        iota = lax.iota(jnp.int32, 16)

        def raw_slice(f, c, g):
            off = pl.multiple_of(c * _CHUNK, _CHUNK) + 16 * g
            return raw_v[f, pl.ds(off, 16)]

        def fill_jbuf(f, c, fh):
            line_base = f * _LINES_PER_VOCAB
            for g in range(_CHUNK // 16):
                codes = raw_slice(f, c, g)
                jbufs[fh][pl.ds(16 * g, 16)] = (
                    lax.shift_right_logical(codes, 2) + line_base)

        def gather_start(f, c, fh):
            fill_jbuf(f, c, fh)
            return pltpu.async_copy(
                tab_hbm.at[jbufs[fh]], stages[fh], gsems[fh])

        def gather_wait(fh):
            pltpu.make_async_copy(
                tab_hbm.at[pl.ds(0, _CHUNK), :], stages[fh],
                gsems[fh]).wait()

        def fixup(f, c, fh, h):
            stage, asm = stages[fh], asms[h]
            col0 = iota * 0 + f * _EMBED_DIM
            for g in range(_CHUNK // 16):
                rows = iota + 16 * g
                s_off = lax.shift_left(
                    lax.bitwise_and(raw_slice(f, c, g), 3), 5)
                for j in range(_EMBED_DIM):
                    vals = plsc.load_gather(stage, [rows, s_off + j])
                    plsc.store_scatter(asm, [rows, col0 + j], vals)

        n_buf = 3

        def chunk_body(c, h):
            for f0 in range(n_buf):
                gather_start(f0, c, f0)

            @pl.loop(0, _NUM_FIELDS + n_buf - 2, step=n_buf)
            def _(g):
                for fh in range(n_buf):
                    f = g + fh

                    @pl.when(f < _NUM_FIELDS)
                    def _():
                        gather_wait(fh)
                        fixup(f, c, fh, h)

                    @pl.when(f + n_buf < _NUM_FIELDS)
                    def _():
                        gather_start(f + n_buf, c, fh)

            row = base + c * _CHUNK
            return pltpu.async_copy(
                asms[h], out_hbm.at[pl.ds(row, _CHUNK), :], wsems[h])

        def write_wait(h):
            pltpu.make_async_copy(
                asms[h], out_hbm.at[pl.ds(0, _CHUNK), :], wsems[h]).wait()

        @pl.loop(0, n_chunks, step=2)
        def _(c):
            for h in range(2):
                @pl.when(c + h >= 2)
                def _():
                    write_wait(h)
                chunk_body(c + h, h)

        for h in range(2):
            write_wait(h)

    return flattener


def kernel(numerical, cat_indices, tables):
    batch = numerical.shape[0]
    tab_lines = tables.reshape(_NUM_FIELDS * _LINES_PER_VOCAB, 128)
    idx_t = cat_indices.astype(jnp.int32).T
    emb = _build(batch)(idx_t, tab_lines)
    return jnp.concatenate([numerical, emb], axis=1)


# speedup vs baseline: 1.0007x; 1.0007x over previous
import functools

import jax
import jax.numpy as jnp
from jax import lax
from jax.experimental import pallas as pl
from jax.experimental.pallas import tpu as pltpu
from jax.experimental.pallas import tpu_sc as plsc

_NUM_FIELDS = 26
_VOCAB = 100000
_EMBED_DIM = 32
_NUM_NUMERICAL = 13
_CHUNK = 32
_LINES_PER_VOCAB = _VOCAB // 4


def _build(batch):
    info = plsc.get_sparse_core_info()
    n_workers = info.num_cores * info.num_subcores
    b_per_w = batch // n_workers
    n_chunks = b_per_w // _CHUNK
    emb_d = _NUM_FIELDS * _EMBED_DIM
    mesh = plsc.VectorSubcoreMesh(core_axis_name="c", subcore_axis_name="s")

    @functools.partial(
        pl.kernel,
        mesh=mesh,
        out_type=jax.ShapeDtypeStruct((batch, emb_d), jnp.float32),
        compiler_params=pltpu.CompilerParams(needs_layout_passes=False, disable_bounds_checks=True),
        scratch_types=[
            pltpu.VMEM((_NUM_FIELDS, b_per_w), jnp.int32),
            pltpu.VMEM((_CHUNK,), jnp.int32),
            pltpu.VMEM((_CHUNK,), jnp.int32),
            pltpu.VMEM((_CHUNK,), jnp.int32),
            pltpu.VMEM((_CHUNK, 128), jnp.float32),
            pltpu.VMEM((_CHUNK, 128), jnp.float32),
            pltpu.VMEM((_CHUNK, 128), jnp.float32),
            pltpu.VMEM((_CHUNK, emb_d), jnp.float32),
            pltpu.VMEM((_CHUNK, emb_d), jnp.float32),
            pltpu.SemaphoreType.DMA,
            pltpu.SemaphoreType.DMA,
            pltpu.SemaphoreType.DMA,
            pltpu.SemaphoreType.DMA,
            pltpu.SemaphoreType.DMA,
        ],
    )
    def flattener(idx_hbm, tab_hbm, out_hbm, raw_v, jbuf0, jbuf1, jbuf2,
                  stage0, stage1, stage2, asm0, asm1,
                  gsem0, gsem1, gsem2, wsem0, wsem1):
        jbufs = (jbuf0, jbuf1, jbuf2)
        stages = (stage0, stage1, stage2)
        asms = (asm0, asm1)
        gsems = (gsem0, gsem1, gsem2)
        wsems = (wsem0, wsem1)
        wid = lax.axis_index("s") * info.num_cores + lax.axis_index("c")
        base = wid * b_per_w

        pltpu.sync_copy(
            idx_hbm.at[:, pl.ds(pl.multiple_of(base, b_per_w), b_per_w)],
            raw_v)

        iota = lax.iota(jnp.int32, 16)

        def raw_slice(f, c, g):
            off = pl.multiple_of(c * _CHUNK, _CHUNK) + 16 * g
            return raw_v[f, pl.ds(off, 16)]

        def fill_jbuf(f, c, fh):
            line_base = f * _LINES_PER_VOCAB
            for g in range(_CHUNK // 16):
                codes = raw_slice(f, c, g)
                jbufs[fh][pl.ds(16 * g, 16)] = (
                    lax.shift_right_logical(codes, 2) + line_base)

        def gather_start(f, c, fh):
            fill_jbuf(f, c, fh)
            return pltpu.async_copy(
                tab_hbm.at[jbufs[fh]], stages[fh], gsems[fh])

        def gather_wait(fh):
            pltpu.make_async_copy(
                tab_hbm.at[pl.ds(0, _CHUNK), :], stages[fh],
                gsems[fh]).wait()

        def fixup(f, c, fh, h):
            stage, asm = stages[fh], asms[h]
            col0 = iota * 0 + f * _EMBED_DIM
            for g in range(_CHUNK // 16):
                rows = iota + 16 * g
                s_off = lax.shift_left(
                    lax.bitwise_and(raw_slice(f, c, g), 3), 5)
                for j in range(_EMBED_DIM):
                    vals = plsc.load_gather(stage, [rows, s_off + j])
                    plsc.store_scatter(asm, [rows, col0 + j], vals)

        n_buf = 3

        def chunk_body(c, h):
            for f0 in range(n_buf):
                gather_start(f0, c, f0)

            @pl.loop(0, _NUM_FIELDS + n_buf - 2, step=n_buf)
            def _(g):
                for fh in range(n_buf):
                    f = g + fh

                    @pl.when(f < _NUM_FIELDS)
                    def _():
                        gather_wait(fh)
                        fixup(f, c, fh, h)

                    @pl.when(f + n_buf < _NUM_FIELDS)
                    def _():
                        gather_start(f + n_buf, c, fh)

            row = base + c * _CHUNK
            return pltpu.async_copy(
                asms[h], out_hbm.at[pl.ds(row, _CHUNK), :], wsems[h])

        def write_wait(h):
            pltpu.make_async_copy(
                asms[h], out_hbm.at[pl.ds(0, _CHUNK), :], wsems[h]).wait()

        @pl.loop(0, n_chunks, step=2)
        def _(c):
            for h in range(2):
                @pl.when(c + h >= 2)
                def _():
                    write_wait(h)
                chunk_body(c + h, h)

        for h in range(2):
            write_wait(h)

    return flattener


def kernel(numerical, cat_indices, tables):
    batch = numerical.shape[0]
    tab_lines = tables.reshape(_NUM_FIELDS * _LINES_PER_VOCAB, 128)
    idx_t = cat_indices.astype(jnp.int32).T
    emb = _build(batch)(idx_t, tab_lines)
    return jnp.concatenate([numerical, emb], axis=1)


# speedup vs baseline: 1.1582x; 1.1574x over previous
import functools

import jax
import jax.numpy as jnp
from jax import lax
from jax.experimental import pallas as pl
from jax.experimental.pallas import tpu as pltpu
from jax.experimental.pallas import tpu_sc as plsc

_NUM_FIELDS = 26
_VOCAB = 100000
_EMBED_DIM = 32
_NUM_NUMERICAL = 13
_LANES = 128
_SUB = 32
_LINES_PER_VOCAB = _VOCAB // 4


def _build(batch):
    info = plsc.get_sparse_core_info()
    n_workers = info.num_cores * info.num_subcores
    b_per_w = batch // n_workers
    n_chunks = b_per_w // _LANES
    out_d = _NUM_NUMERICAL + _NUM_FIELDS * _EMBED_DIM
    mesh = plsc.VectorSubcoreMesh(core_axis_name="c", subcore_axis_name="s")

    @functools.partial(
        pl.kernel,
        mesh=mesh,
        out_type=jax.ShapeDtypeStruct((out_d, batch), jnp.float32),
        compiler_params=pltpu.CompilerParams(
            needs_layout_passes=False, disable_bounds_checks=True),
        scratch_types=[
            pltpu.VMEM((_NUM_FIELDS, _LANES), jnp.int32),
            pltpu.VMEM((_NUM_NUMERICAL, _LANES), jnp.float32),
            pltpu.VMEM((_SUB,), jnp.int32),
            pltpu.VMEM((_SUB,), jnp.int32),
            pltpu.VMEM((_SUB, 128), jnp.float32),
            pltpu.VMEM((_SUB, 128), jnp.float32),
            pltpu.VMEM((out_d, _LANES), jnp.float32),
            pltpu.SemaphoreType.DMA,
            pltpu.SemaphoreType.DMA,
            pltpu.SemaphoreType.DMA,
        ],
    )
    def flattener(num_hbm, idx_hbm, tab_hbm, out_hbm, rawc, nstg,
                  jbuf0, jbuf1, stg0, stg1, asm, gsem0, gsem1, wsem):
        jbufs = (jbuf0, jbuf1)
        stgs = (stg0, stg1)
        gsems = (gsem0, gsem1)
        wid = lax.axis_index("s") * info.num_cores + lax.axis_index("c")
        lane_base = wid * b_per_w
        iota = lax.iota(jnp.int32, 16)
        n_sub = _LANES // _SUB

        def codes(f, s, g):
            return rawc[f, pl.ds(_SUB * s + 16 * g, 16)]

        def gather_start(f, s):
            b = s % 2
            line_base = f * _LINES_PER_VOCAB
            for g in range(_SUB // 16):
                jbufs[b][pl.ds(16 * g, 16)] = (
                    lax.shift_right_logical(codes(f, s, g), 2) + line_base)
            return pltpu.async_copy(tab_hbm.at[jbufs[b]], stgs[b], gsems[b])

        def gather_wait(b):
            pltpu.make_async_copy(
                tab_hbm.at[pl.ds(0, _SUB), :], stgs[b], gsems[b]).wait()

        def fixup(f, s, b):
            row0 = _NUM_NUMERICAL + f * _EMBED_DIM
            for g in range(_SUB // 16):
                rows = iota + 16 * g
                s_off = lax.shift_left(lax.bitwise_and(codes(f, s, g), 3), 5)
                for d in range(_EMBED_DIM):
                    vals = plsc.load_gather(stgs[b], [rows, s_off + d])
                    asm[row0 + d, pl.ds(_SUB * s + 16 * g, 16)] = vals

        def write_wait():
            pltpu.make_async_copy(
                asm, out_hbm.at[:, pl.ds(0, _LANES)], wsem).wait()

        @pl.loop(0, n_chunks)
        def _(c):
            lane0 = pl.multiple_of(lane_base + c * _LANES, _LANES)
            pltpu.sync_copy(idx_hbm.at[:, pl.ds(lane0, _LANES)], rawc)
            pltpu.sync_copy(num_hbm.at[:, pl.ds(lane0, _LANES)], nstg)

            @pl.when(c >= 1)
            def _():
                write_wait()

            for d in range(_NUM_NUMERICAL):
                for g in range(_LANES // 16):
                    asm[d, pl.ds(16 * g, 16)] = nstg[d, pl.ds(16 * g, 16)]

            gather_start(0, 0)
            gather_start(0, 1)

            @pl.loop(0, _NUM_FIELDS)
            def _(f):
                for s in range(n_sub):
                    b = s % 2
                    gather_wait(b)
                    fixup(f, s, b)
                    if s + 2 < n_sub:
                        gather_start(f, s + 2)
                    else:
                        @pl.when(f + 1 < _NUM_FIELDS)
                        def _():
                            gather_start(f + 1, s + 2 - n_sub)

            pltpu.async_copy(asm, out_hbm.at[:, pl.ds(lane0, _LANES)], wsem)

        write_wait()

    return flattener


def kernel(numerical, cat_indices, tables):
    batch = numerical.shape[0]
    tab_lines = tables.reshape(_NUM_FIELDS * _LINES_PER_VOCAB, 128)
    idx_t = cat_indices.astype(jnp.int32).T
    num_t = numerical.T
    out_t = _build(batch)(num_t, idx_t, tab_lines)
    return out_t.T


# speedup vs baseline: 1.2210x; 1.0543x over previous
import functools

import jax
import jax.numpy as jnp
from jax import lax
from jax.experimental import pallas as pl
from jax.experimental.pallas import tpu as pltpu
from jax.experimental.pallas import tpu_sc as plsc

_NUM_FIELDS = 26
_VOCAB = 100000
_EMBED_DIM = 32
_NUM_NUMERICAL = 13
_LANES = 128
_SUB = 32
_LINES_PER_VOCAB = _VOCAB // 4


def _build(batch):
    info = plsc.get_sparse_core_info()
    n_workers = info.num_cores * info.num_subcores
    b_per_w = batch // n_workers
    n_chunks = b_per_w // _LANES
    out_d = _NUM_NUMERICAL + _NUM_FIELDS * _EMBED_DIM
    mesh = plsc.VectorSubcoreMesh(core_axis_name="c", subcore_axis_name="s")

    @functools.partial(
        pl.kernel,
        mesh=mesh,
        out_type=jax.ShapeDtypeStruct((out_d, batch), jnp.float32),
        compiler_params=pltpu.CompilerParams(
            needs_layout_passes=False, disable_bounds_checks=True),
        scratch_types=[
            pltpu.VMEM((_NUM_FIELDS, _LANES), jnp.int32),
            pltpu.VMEM((_NUM_NUMERICAL, _LANES), jnp.float32),
            pltpu.VMEM((_SUB,), jnp.int32),
            pltpu.VMEM((_SUB,), jnp.int32),
            pltpu.VMEM((_SUB, 128), jnp.float32),
            pltpu.VMEM((_SUB, 128), jnp.float32),
            pltpu.VMEM((out_d, _LANES), jnp.float32),
            pltpu.SemaphoreType.DMA,
            pltpu.SemaphoreType.DMA,
            pltpu.SemaphoreType.DMA,
        ],
    )
    def flattener(num_hbm, idx_hbm, tab_hbm, out_hbm, rawc, nstg,
                  jbuf0, jbuf1, stg0, stg1, asm, gsem0, gsem1, wsem):
        jbufs = (jbuf0, jbuf1)
        stgs = (stg0, stg1)
        gsems = (gsem0, gsem1)
        wid = lax.axis_index("s") * info.num_cores + lax.axis_index("c")
        lane_base = wid * b_per_w
        iota = lax.iota(jnp.int32, 16)
        n_sub = _LANES // _SUB

        def codes(f, s, g):
            return rawc[f, pl.ds(_SUB * s + 16 * g, 16)]

        def gather_start(f, s):
            b = s % 2
            line_base = f * _LINES_PER_VOCAB
            for g in range(_SUB // 16):
                jbufs[b][pl.ds(16 * g, 16)] = (
                    lax.shift_right_logical(codes(f, s, g), 2) + line_base)
            return pltpu.async_copy(tab_hbm.at[jbufs[b]], stgs[b], gsems[b])

        def gather_wait(b):
            pltpu.make_async_copy(
                tab_hbm.at[pl.ds(0, _SUB), :], stgs[b], gsems[b]).wait()

        def fixup(f, s, b):
            row0 = _NUM_NUMERICAL + f * _EMBED_DIM
            for g in range(_SUB // 16):
                rows = iota + 16 * g
                s_off = lax.shift_left(lax.bitwise_and(codes(f, s, g), 3), 5)

                @plsc.parallel_loop(0, _EMBED_DIM, step=1, unroll=8)
                def _(d):
                    vals = plsc.load_gather(stgs[b], [rows, s_off + d])
                    asm[row0 + d, pl.ds(_SUB * s + 16 * g, 16)] = vals

        def write_wait():
            pltpu.make_async_copy(
                asm, out_hbm.at[:, pl.ds(0, _LANES)], wsem).wait()

        @pl.loop(0, n_chunks)
        def _(c):
            lane0 = pl.multiple_of(lane_base + c * _LANES, _LANES)
            pltpu.sync_copy(idx_hbm.at[:, pl.ds(lane0, _LANES)], rawc)
            pltpu.sync_copy(num_hbm.at[:, pl.ds(lane0, _LANES)], nstg)

            @pl.when(c >= 1)
            def _():
                write_wait()

            for d in range(_NUM_NUMERICAL):
                for g in range(_LANES // 16):
                    asm[d, pl.ds(16 * g, 16)] = nstg[d, pl.ds(16 * g, 16)]

            gather_start(0, 0)
            gather_start(0, 1)

            @pl.loop(0, _NUM_FIELDS)
            def _(f):
                for s in range(n_sub):
                    b = s % 2
                    gather_wait(b)
                    fixup(f, s, b)
                    if s + 2 < n_sub:
                        gather_start(f, s + 2)
                    else:
                        @pl.when(f + 1 < _NUM_FIELDS)
                        def _():
                            gather_start(f + 1, s + 2 - n_sub)

            pltpu.async_copy(asm, out_hbm.at[:, pl.ds(lane0, _LANES)], wsem)

        write_wait()

    return flattener


def kernel(numerical, cat_indices, tables):
    batch = numerical.shape[0]
    tab_lines = tables.reshape(_NUM_FIELDS * _LINES_PER_VOCAB, 128)
    idx_t = cat_indices.astype(jnp.int32).T
    num_t = numerical.T
    out_t = _build(batch)(num_t, idx_t, tab_lines)
    return out_t.T
